# baseline (speedup 1.0000x reference)
import jax
import jax.numpy as jnp
from jax.experimental import pallas as pl

NUM_CODES = 8192
DIM = 32
TILE_M = 512


def _normalize_kernel(emb_ref, out_ref):
    e = emb_ref[...]
    n = jnp.sqrt(jnp.sum(e * e, axis=1, keepdims=True))
    out_ref[...] = e / jnp.maximum(n, 1e-12)


def _vq_kernel(flat_ref, en_ref, emb_ref, idx_ref, zq_ref):
    x = flat_ref[...]
    n = jnp.sqrt(jnp.sum(x * x, axis=1, keepdims=True))
    ez = x / jnp.maximum(n, 1e-12)
    en = en_ref[...]
    sim = jax.lax.dot_general(ez, en, (((1,), (1,)), ((), ())))
    idx = jnp.argmax(sim, axis=1).astype(jnp.int32)
    idx_ref[0, 0, :] = idx
    onehot = (jax.lax.broadcasted_iota(jnp.int32, (TILE_M, NUM_CODES), 1)
              == idx[:, None]).astype(jnp.float32)
    zq_ref[...] = jax.lax.dot_general(
        onehot, emb_ref[...], (((1,), (0,)), ((), ())),
        precision=jax.lax.Precision.HIGHEST)


def kernel(z, embedding):
    b, d, tq = z.shape
    m = b * tq
    flat = jnp.transpose(z, (0, 2, 1)).reshape(m, d)

    en = pl.pallas_call(
        _normalize_kernel,
        out_shape=jax.ShapeDtypeStruct((NUM_CODES, DIM), jnp.float32),
    )(embedding)

    grid = (m // TILE_M,)
    idx3, zqf = pl.pallas_call(
        _vq_kernel,
        grid=grid,
        in_specs=[
            pl.BlockSpec((TILE_M, DIM), lambda i: (i, 0)),
            pl.BlockSpec((NUM_CODES, DIM), lambda i: (0, 0)),
            pl.BlockSpec((NUM_CODES, DIM), lambda i: (0, 0)),
        ],
        out_specs=[
            pl.BlockSpec((1, 1, TILE_M), lambda i: (i, 0, 0)),
            pl.BlockSpec((TILE_M, DIM), lambda i: (i, 0)),
        ],
        out_shape=[
            jax.ShapeDtypeStruct((m // TILE_M, 1, TILE_M), jnp.int32),
            jax.ShapeDtypeStruct((m, DIM), jnp.float32),
        ],
    )(flat, en, embedding)

    idx = idx3.reshape(b, tq)
    z_q = jnp.transpose(zqf.reshape(b, tq, d), (0, 2, 1))
    z_q_st = z + jax.lax.stop_gradient(z_q - z)
    return (z_q_st, idx, z_q)


# speedup vs baseline: 5.1197x; 5.1197x over previous
import functools

import jax
import jax.numpy as jnp
from jax import lax
from jax.experimental import pallas as pl
from jax.experimental.pallas import tpu as pltpu
from jax.experimental.pallas import tpu_sc as plsc

NUM_CODES = 8192
DIM = 32
TILE_M = 512

_SC_INFO = plsc.get_sparse_core_info()
_NC = _SC_INFO.num_cores
_NS = _SC_INFO.num_subcores
_NW = _NC * _NS
_CHUNK = 128


def _normalize_kernel(emb_ref, out_ref):
    e = emb_ref[...]
    n = jnp.sqrt(jnp.sum(e * e, axis=1, keepdims=True))
    out_ref[...] = e / jnp.maximum(n, 1e-12)


def _vq_kernel(flat_ref, en_ref, idx_ref):
    x = flat_ref[...]
    n = jnp.sqrt(jnp.sum(x * x, axis=1, keepdims=True))
    ez = x / jnp.maximum(n, 1e-12)
    en = en_ref[...]
    sim = jax.lax.dot_general(ez, en, (((1,), (1,)), ((), ())))
    idx_ref[0, 0, :] = jnp.argmax(sim, axis=1).astype(jnp.int32)


def _make_sc_gather(b_total):
    b_per_w = b_total // _NW
    n_chunks = b_per_w // _CHUNK
    mesh = plsc.VectorSubcoreMesh(core_axis_name="c", subcore_axis_name="s")

    @functools.partial(
        pl.kernel,
        mesh=mesh,
        compiler_params=pltpu.CompilerParams(use_tc_tiling_on_sc=False),
        out_type=jax.ShapeDtypeStruct((b_total, DIM), jnp.float32),
        scratch_types=[
            pltpu.VMEM((n_chunks, _CHUNK), jnp.int32),
            pltpu.VMEM((b_per_w, DIM), jnp.float32),
            pltpu.SemaphoreType.DMA,
        ],
    )
    def sc_gather(table_hbm, idx_hbm, out_hbm, idx_v, rows_v, sem):
        wid = lax.axis_index("s") * _NC + lax.axis_index("c")
        base = wid * b_per_w
        pltpu.sync_copy(idx_hbm.at[wid], idx_v)
        for j in range(n_chunks):
            pltpu.async_copy(
                table_hbm.at[idx_v.at[j]],
                rows_v.at[pl.ds(j * _CHUNK, _CHUNK)],
                sem,
            )
        for j in range(n_chunks):
            pltpu.make_async_copy(
                table_hbm.at[idx_v.at[j]],
                rows_v.at[pl.ds(j * _CHUNK, _CHUNK)],
                sem,
            ).wait()
        pltpu.sync_copy(rows_v, out_hbm.at[pl.ds(base, b_per_w)])

    return sc_gather


def kernel(z, embedding):
    b, d, tq = z.shape
    m = b * tq
    flat = jnp.transpose(z, (0, 2, 1)).reshape(m, d)

    en = pl.pallas_call(
        _normalize_kernel,
        out_shape=jax.ShapeDtypeStruct((NUM_CODES, DIM), jnp.float32),
    )(embedding)

    grid = (m // TILE_M,)
    idx3 = pl.pallas_call(
        _vq_kernel,
        grid=grid,
        in_specs=[
            pl.BlockSpec((TILE_M, DIM), lambda i: (i, 0)),
            pl.BlockSpec((NUM_CODES, DIM), lambda i: (0, 0)),
        ],
        out_specs=pl.BlockSpec((1, 1, TILE_M), lambda i: (i, 0, 0)),
        out_shape=jax.ShapeDtypeStruct((m // TILE_M, 1, TILE_M), jnp.int32),
    )(flat, en)

    idx_flat = idx3.reshape(m)
    idx_w = idx_flat.reshape(_NW, m // (_NW * _CHUNK), _CHUNK)
    zqf = _make_sc_gather(m)(embedding, idx_w)

    idx = idx_flat.reshape(b, tq)
    z_q = jnp.transpose(zqf.reshape(b, tq, d), (0, 2, 1))
    z_q_st = z + jax.lax.stop_gradient(z_q - z)
    return (z_q_st, idx, z_q)
